# baseline (speedup 1.0000x reference)
import jax
import jax.numpy as jnp
from jax.experimental import pallas as pl
from jax.experimental.pallas import tpu as pltpu

B, D, T = 16, 64, 2048
K = 1024
N = B * T
TB = 256
NB = N // TB
NH = 2


def _rvq_kernel(x_ref, cb0_ref, cb1_ref, cb2_ref,
                cc0_ref, cc1_ref, cc2_ref,
                p0_ref, p1_ref,
                idx_ref, loss_ref, codes_scr):
    step = pl.program_id(0)
    xb = x_ref[0]
    zfull = xb.T
    HB = TB // NH

    total = jnp.float32(0.0)
    zs = [zfull[h * HB:(h + 1) * HB, :] for h in range(NH)]
    zzs = [jnp.sum(zh * zh, axis=1, keepdims=True) for zh in zs]
    idxs = [None] * NH
    for q, (cb_ref, cc_ref) in enumerate(((cb0_ref, cc0_ref),
                                          (cb1_ref, cc1_ref),
                                          (cb2_ref, cc2_ref))):
        cb = cb_ref[...]
        cc = cc_ref[...]
        dims = (((1,), (0,)), ((), ()))
        for h in range(NH):
            z, zz = zs[h], zzs[h]
            m = jax.lax.dot_general(z, cb, (((1,), (1,)), ((), ())),
                                    preferred_element_type=jnp.float32)
            d = zz - 2.0 * m + cc
            if q < 2:
                p = (p0_ref, p1_ref)[q]
                dmin = jnp.min(d, axis=1, keepdims=True)
                mask = d == dmin
                maskb = mask.astype(jnp.bfloat16)
                g = jax.lax.dot_general(maskb, p[...], dims,
                                        preferred_element_type=jnp.float32)
                row = pl.ds(h * HB, HB)
                codes_scr[row, :] = ((g[:, 0:D] + g[:, D:2 * D])
                                     + g[:, 2 * D:3 * D])
                anytie = jnp.max(g[:, 3 * D:3 * D + 1]) >= 1.5

                @pl.when(anytie)
                def _(mask=mask, d=d, p=p, row=row):
                    iota = jax.lax.broadcasted_iota(jnp.int32, d.shape, 1)
                    idxvec = jnp.where(mask, iota, K)
                    first = jnp.min(idxvec, axis=1)
                    onehot = (idxvec == first[:, None]).astype(jnp.bfloat16)
                    g2 = jax.lax.dot_general(onehot, p[...], dims,
                                             preferred_element_type=jnp.float32)
                    codes_scr[row, :] = ((g2[:, 0:D] + g2[:, D:2 * D])
                                         + g2[:, 2 * D:3 * D])

                zs[h] = z - codes_scr[row, :]
                zzs[h] = jnp.sum(zs[h] * zs[h], axis=1, keepdims=True)
                total = total + jnp.sum(zzs[h])
            else:
                idxs[h] = jnp.argmin(d, axis=1)
                dmin = jnp.min(d, axis=1, keepdims=True)
                total = total + jnp.sum(dmin)

    idx_ref[0, 0, :] = jnp.concatenate(idxs, axis=0)

    @pl.when(step == 0)
    def _():
        loss_ref[:, :] = jnp.zeros((1, 1), jnp.float32)

    loss_ref[:, :] += total


def _packed_table(cb):
    hi = cb.astype(jnp.bfloat16)
    r1 = cb - hi.astype(jnp.float32)
    mid = r1.astype(jnp.bfloat16)
    lo = (r1 - mid.astype(jnp.float32)).astype(jnp.bfloat16)
    ones = jnp.ones((K, 1), jnp.bfloat16)
    pad = jnp.zeros((K, 256 - 3 * D - 1), jnp.bfloat16)
    return jnp.concatenate([hi, mid, lo, ones, pad], axis=1)


def kernel(x, cb0, cb1, cb2):
    nT = T // TB

    idx_blocks, loss_sum = pl.pallas_call(
        _rvq_kernel,
        grid=(NB,),
        in_specs=[
            pl.BlockSpec((1, D, TB), lambda i: (i // nT, 0, i % nT)),
            pl.BlockSpec((K, D), lambda i: (0, 0)),
            pl.BlockSpec((K, D), lambda i: (0, 0)),
            pl.BlockSpec((K, D), lambda i: (0, 0)),
        ] + [pl.BlockSpec((1, K), lambda i: (0, 0)) for _ in range(3)]
          + [pl.BlockSpec((K, 256), lambda i: (0, 0)) for _ in range(2)],
        out_specs=[
            pl.BlockSpec((1, 1, TB), lambda i: (i, 0, 0)),
            pl.BlockSpec((1, 1), lambda i: (0, 0)),
        ],
        out_shape=[
            jax.ShapeDtypeStruct((NB, 1, TB), jnp.int32),
            jax.ShapeDtypeStruct((1, 1), jnp.float32),
        ],
        scratch_shapes=[pltpu.VMEM((TB, D), jnp.float32)],
    )(x, cb0, cb1, cb2,
      jnp.sum(cb0 * cb0, axis=1)[None, :],
      jnp.sum(cb1 * cb1, axis=1)[None, :],
      jnp.sum(cb2 * cb2, axis=1)[None, :],
      _packed_table(cb0), _packed_table(cb1))

    code_index = idx_blocks.reshape(B, T)
    loss = (loss_sum[0, 0] / jnp.float32(N * D)).astype(jnp.float32)
    loss = loss.reshape(())
    return (code_index, loss, loss)


# speedup vs baseline: 1.4147x; 1.4147x over previous
import jax
import jax.numpy as jnp
from jax.experimental import pallas as pl

B, D, T = 16, 64, 2048
K = 1024
N = B * T
TB = 256
NB = N // TB


def _rvq_kernel(x_ref, cb0_ref, cb1_ref, cb2_ref,
                cc0_ref, cc1_ref, cc2_ref,
                h0_ref, m0_ref, l0_ref, h1_ref, m1_ref, l1_ref,
                idx_ref, loss_ref):
    step = pl.program_id(0)
    xb = x_ref[0]
    z = xb.T

    total = jnp.float32(0.0)
    idx = None
    zz = jnp.sum(z * z, axis=1, keepdims=True)
    for q, (cb_ref, cc_ref) in enumerate(((cb0_ref, cc0_ref),
                                          (cb1_ref, cc1_ref),
                                          (cb2_ref, cc2_ref))):
        cb = cb_ref[...]
        cc = cc_ref[...]
        m = jax.lax.dot_general(z, cb, (((1,), (1,)), ((), ())),
                                preferred_element_type=jnp.float32)
        d = zz - 2.0 * m + cc
        idx = jnp.argmin(d, axis=1)
        if q < 2:
            h, m_, l = ((h0_ref, m0_ref, l0_ref),
                        (h1_ref, m1_ref, l1_ref))[q]
            iota = jax.lax.broadcasted_iota(jnp.int32, d.shape, 1)
            onehot = (iota == idx[:, None]).astype(jnp.bfloat16)
            dims = (((1,), (0,)), ((), ()))
            codes = (jax.lax.dot_general(onehot, h[...], dims,
                                         preferred_element_type=jnp.float32)
                     + jax.lax.dot_general(onehot, m_[...], dims,
                                           preferred_element_type=jnp.float32)
                     + jax.lax.dot_general(onehot, l[...], dims,
                                           preferred_element_type=jnp.float32))
            z = z - codes
            zz = jnp.sum(z * z, axis=1, keepdims=True)
            total = total + jnp.sum(zz)
        else:
            dmin = jnp.min(d, axis=1, keepdims=True)
            total = total + jnp.sum(dmin)

    idx_ref[0, 0, :] = idx

    @pl.when(step == 0)
    def _():
        loss_ref[:, :] = jnp.zeros((1, 1), jnp.float32)

    loss_ref[:, :] += total


def _split3(cb):
    hi = cb.astype(jnp.bfloat16)
    r1 = cb - hi.astype(jnp.float32)
    mid = r1.astype(jnp.bfloat16)
    lo = (r1 - mid.astype(jnp.float32)).astype(jnp.bfloat16)
    return hi, mid, lo


def kernel(x, cb0, cb1, cb2):
    nT = T // TB

    idx_blocks, loss_sum = pl.pallas_call(
        _rvq_kernel,
        grid=(NB,),
        in_specs=[
            pl.BlockSpec((1, D, TB), lambda i: (i // nT, 0, i % nT)),
            pl.BlockSpec((K, D), lambda i: (0, 0)),
            pl.BlockSpec((K, D), lambda i: (0, 0)),
            pl.BlockSpec((K, D), lambda i: (0, 0)),
        ] + [pl.BlockSpec((1, K), lambda i: (0, 0)) for _ in range(3)]
          + [pl.BlockSpec((K, D), lambda i: (0, 0)) for _ in range(6)],
        out_specs=[
            pl.BlockSpec((1, 1, TB), lambda i: (i, 0, 0)),
            pl.BlockSpec((1, 1), lambda i: (0, 0)),
        ],
        out_shape=[
            jax.ShapeDtypeStruct((NB, 1, TB), jnp.int32),
            jax.ShapeDtypeStruct((1, 1), jnp.float32),
        ],
    )(x, cb0, cb1, cb2,
      jnp.sum(cb0 * cb0, axis=1)[None, :],
      jnp.sum(cb1 * cb1, axis=1)[None, :],
      jnp.sum(cb2 * cb2, axis=1)[None, :],
      *_split3(cb0), *_split3(cb1))

    code_index = idx_blocks.reshape(B, T)
    loss = (loss_sum[0, 0] / jnp.float32(N * D)).astype(jnp.float32)
    loss = loss.reshape(())
    return (code_index, loss, loss)
